# Initial kernel scaffold; baseline (speedup 1.0000x reference)
import jax
import jax.numpy as jnp
from jax.experimental import pallas as pl


def kernel(x, edge_index, batch, fc11_w, fc11_b, fc12_w, fc12_b, fc13_w, fc13_b, conv11_w, conv11_b, bn1_g, bn1_b, fc21_w, fc21_b, fc22_w, fc22_b, fc23_w, fc23_b, conv21_w, conv21_b, bn2_g, bn2_b, fc31_w, fc31_b, fc32_w, fc32_b, fc33_w, fc33_b, conv31_w, conv31_b, bn3_g, bn3_b, fc1_w, fc1_b, fc2_w, fc2_b):
    raise NotImplementedError("write your pallas kernel here")


# speedup vs baseline: 8.1805x; 8.1805x over previous
import functools

import jax
import jax.numpy as jnp
from jax import lax
from jax.experimental import pallas as pl
from jax.experimental.pallas import tpu as pltpu
from jax.experimental.pallas import tpu_sc as plsc

N = 10000
E = 320000
NG = 64
F = 32
NC, NS = 2, 16
NW = NC * NS
CHUNK = 128
E_PAD = ((E + NW * CHUNK - 1) // (NW * CHUNK)) * (NW * CHUNK)
CPT = E_PAD // (NW * CHUNK)
N_ACC = 10112
DUMMY = N
RPT = N_ACC // NS


def _sc_scatter_add(table, src3, dst3, zeros):
    mesh = plsc.VectorSubcoreMesh(core_axis_name="c", subcore_axis_name="s")

    @functools.partial(
        pl.kernel,
        out_type=jax.ShapeDtypeStruct((NC, N_ACC, F), jnp.float32),
        mesh=mesh,
        scratch_types=[
            pltpu.VMEM_SHARED((N_ACC, F), jnp.float32),
            pltpu.VMEM((CPT, CHUNK), jnp.int32),
            pltpu.VMEM((CPT, CHUNK), jnp.int32),
            pltpu.VMEM((CHUNK, F), jnp.float32),
            pltpu.SemaphoreType.DMA,
        ],
        compiler_params=pltpu.CompilerParams(use_tc_tiling_on_sc=False),
    )
    def k(table_hbm, src_hbm, dst_hbm, zero_hbm, out_hbm,
          acc, src_v, dst_v, rows_v, sem):
        c = lax.axis_index("c")
        s = lax.axis_index("s")
        glob = c * NS + s
        pltpu.sync_copy(zero_hbm.at[pl.ds(s * RPT, RPT)],
                        acc.at[pl.ds(s * RPT, RPT)])
        pltpu.sync_copy(src_hbm.at[glob], src_v)
        pltpu.sync_copy(dst_hbm.at[glob], dst_v)
        plsc.subcore_barrier()

        def step(j, carry):
            pltpu.async_copy(table_hbm.at[src_v.at[j]], rows_v, sem).wait()
            pltpu.sync_copy(rows_v, acc.at[dst_v.at[j]], add=True)
            return carry

        lax.fori_loop(0, CPT, step, 0)
        plsc.subcore_barrier()
        pltpu.sync_copy(acc.at[pl.ds(s * RPT, RPT)],
                        out_hbm.at[c, pl.ds(s * RPT, RPT)])

    return k(table, src3, dst3, zeros)


def _relu(t):
    return jnp.maximum(t, 0.0)


def _dot(a, b):
    return jnp.dot(a, b, preferred_element_type=jnp.float32)


def _tc_first(x, w1, b1, w2, b2, w3, b3, cw):
    def body(x_r, w1_r, b1_r, w2_r, b2_r, w3_r, b3_r, cw_r,
             fca_r, fbc_r, hw_r):
        xv = x_r[...]
        fca_r[...] = _relu(_dot(xv, w1_r[...]) + b1_r[...])
        fbc_r[...] = (_relu(_dot(xv, w2_r[...]) + b2_r[...]) *
                      _relu(_dot(xv, w3_r[...]) + b3_r[...]))
        hw_r[...] = _dot(xv, cw_r[...])

    return pl.pallas_call(
        body,
        out_shape=[
            jax.ShapeDtypeStruct((N, 16), jnp.float32),
            jax.ShapeDtypeStruct((N, 16), jnp.float32),
            jax.ShapeDtypeStruct((N, F), jnp.float32),
        ],
    )(x, w1, b1, w2, b2, w3, b3, cw)


def _finish_layer(p0_r, p1_r, cb_r, fca_r, fbc_r, g_r, b_r):
    conv = _relu(p0_r[...] + p1_r[...] + cb_r[...])
    h = jnp.concatenate([fca_r[...], conv, fbc_r[...]], axis=1)
    m = jnp.mean(h, axis=0, keepdims=True)
    v = jnp.mean((h - m) * (h - m), axis=0, keepdims=True)
    return (h - m) * lax.rsqrt(v + 1e-5) * g_r[...] + b_r[...]


def _tc_mid(p0, p1, cb, fca, fbc, g, b, w1, b1, w2, b2, w3, b3, cw):
    def body(p0_r, p1_r, cb_r, fca_r, fbc_r, g_r, b_r,
             w1_r, b1_r, w2_r, b2_r, w3_r, b3_r, cw_r,
             fca_o, fbc_o, hw_o):
        h = _finish_layer(p0_r, p1_r, cb_r, fca_r, fbc_r, g_r, b_r)
        fca_o[...] = _relu(_dot(h, w1_r[...]) + b1_r[...])
        fbc_o[...] = (_relu(_dot(h, w2_r[...]) + b2_r[...]) *
                      _relu(_dot(h, w3_r[...]) + b3_r[...]))
        hw_o[...] = _dot(h, cw_r[...])

    return pl.pallas_call(
        body,
        out_shape=[
            jax.ShapeDtypeStruct((N, 16), jnp.float32),
            jax.ShapeDtypeStruct((N, 16), jnp.float32),
            jax.ShapeDtypeStruct((N, F), jnp.float32),
        ],
    )(p0, p1, cb, fca, fbc, g, b, w1, b1, w2, b2, w3, b3, cw)


def _tc_final(p0, p1, cb, fca, fbc, g, b, batch2d, fc1_w, fc1_b, fc2_w, fc2_b):
    def body(p0_r, p1_r, cb_r, fca_r, fbc_r, g_r, b_r, bt_r,
             w1_r, b1_r, w2_r, b2_r, out_r):
        h = _finish_layer(p0_r, p1_r, cb_r, fca_r, fbc_r, g_r, b_r)
        seg = lax.broadcasted_iota(jnp.int32, (N, NG), 1)
        oh = (bt_r[...] == seg).astype(jnp.float32)
        sums = lax.dot_general(oh, h, (((0,), (0,)), ((), ())),
                               preferred_element_type=jnp.float32)
        cnt = jnp.sum(oh, axis=0)
        pooled = sums / jnp.maximum(cnt, 1.0)[:, None]
        hid = _relu(_dot(pooled, w1_r[...]) + b1_r[...])
        out_r[...] = _dot(hid, w2_r[...]) + b2_r[...]

    return pl.pallas_call(
        body,
        out_shape=jax.ShapeDtypeStruct((NG, 1), jnp.float32),
    )(p0, p1, cb, fca, fbc, g, b, batch2d, fc1_w, fc1_b, fc2_w, fc2_b)


def kernel(x, edge_index, batch,
           fc11_w, fc11_b, fc12_w, fc12_b, fc13_w, fc13_b, conv11_w, conv11_b,
           bn1_g, bn1_b,
           fc21_w, fc21_b, fc22_w, fc22_b, fc23_w, fc23_b, conv21_w, conv21_b,
           bn2_g, bn2_b,
           fc31_w, fc31_b, fc32_w, fc32_b, fc33_w, fc33_b, conv31_w, conv31_b,
           bn3_g, bn3_b,
           fc1_w, fc1_b, fc2_w, fc2_b):
    r2 = lambda t: t[None, :]

    pad = E_PAD - E
    src = jnp.concatenate([edge_index[0], jnp.zeros((pad,), jnp.int32)])
    dst = jnp.concatenate([edge_index[1], jnp.full((pad,), DUMMY, jnp.int32)])
    src3 = src.reshape(NW, CPT, CHUNK)
    dst3 = dst.reshape(NW, CPT, CHUNK)
    zeros = jnp.zeros((N_ACC, F), jnp.float32)
    batch2d = batch.reshape(N, 1)

    fca, fbc, hw = _tc_first(x, fc11_w, r2(fc11_b), fc12_w, r2(fc12_b),
                             fc13_w, r2(fc13_b), conv11_w)
    agg = _sc_scatter_add(hw, src3, dst3, zeros)
    p0, p1 = agg[0, :N], agg[1, :N]

    fca, fbc, hw = _tc_mid(p0, p1, r2(conv11_b), fca, fbc, r2(bn1_g), r2(bn1_b),
                           fc21_w, r2(fc21_b), fc22_w, r2(fc22_b),
                           fc23_w, r2(fc23_b), conv21_w)
    agg = _sc_scatter_add(hw, src3, dst3, zeros)
    p0, p1 = agg[0, :N], agg[1, :N]

    fca, fbc, hw = _tc_mid(p0, p1, r2(conv21_b), fca, fbc, r2(bn2_g), r2(bn2_b),
                           fc31_w, r2(fc31_b), fc32_w, r2(fc32_b),
                           fc33_w, r2(fc33_b), conv31_w)
    agg = _sc_scatter_add(hw, src3, dst3, zeros)
    p0, p1 = agg[0, :N], agg[1, :N]

    return _tc_final(p0, p1, r2(conv31_b), fca, fbc, r2(bn3_g), r2(bn3_b),
                     batch2d, fc1_w, r2(fc1_b), fc2_w, r2(fc2_b))


# speedup vs baseline: 8.2427x; 1.0076x over previous
import functools

import jax
import jax.numpy as jnp
from jax import lax
from jax.experimental import pallas as pl
from jax.experimental.pallas import tpu as pltpu
from jax.experimental.pallas import tpu_sc as plsc

N = 10000
E = 320000
NG = 64
F = 32
NC, NS = 2, 16
NW = NC * NS
CHUNK = 128
GRP = 4
CPT = 80
E_PAD = NW * CHUNK * CPT
N_ACC = 10112
DUMMY = N
RPT = N_ACC // NS


def _sc_scatter_add(table, src3, dst3, zeros):
    mesh = plsc.VectorSubcoreMesh(core_axis_name="c", subcore_axis_name="s")

    @functools.partial(
        pl.kernel,
        out_type=jax.ShapeDtypeStruct((NC, N_ACC, F), jnp.float32),
        mesh=mesh,
        scratch_types=[
            pltpu.VMEM_SHARED((N_ACC, F), jnp.float32),
            pltpu.VMEM((CPT, CHUNK), jnp.int32),
            pltpu.VMEM((CPT, CHUNK), jnp.int32),
            pltpu.VMEM((2 * GRP, CHUNK, F), jnp.float32),
            [pltpu.SemaphoreType.DMA] * (2 * GRP),
            [pltpu.SemaphoreType.DMA] * (2 * GRP),
        ],
        compiler_params=pltpu.CompilerParams(use_tc_tiling_on_sc=False),
    )
    def k(table_hbm, src_hbm, dst_hbm, zero_hbm, out_hbm,
          acc, src_v, dst_v, rows_v, sg, ss):
        c = lax.axis_index("c")
        s = lax.axis_index("s")
        glob = c * NS + s
        pltpu.sync_copy(zero_hbm.at[pl.ds(s * RPT, RPT)],
                        acc.at[pl.ds(s * RPT, RPT)])
        pltpu.sync_copy(src_hbm.at[glob], src_v)
        pltpu.sync_copy(dst_hbm.at[glob], dst_v)
        plsc.subcore_barrier()

        def fire_gather(j, b):
            pltpu.async_copy(table_hbm.at[src_v.at[j]], rows_v.at[b], sg[b])

        def wait_gather(b):
            pltpu.make_async_copy(table_hbm.at[src_v.at[0]],
                                  rows_v.at[b], sg[b]).wait()

        def fire_scatter(j, b):
            pltpu.async_copy(rows_v.at[b], acc.at[dst_v.at[j]], ss[b],
                             add=True)

        def wait_scatter(b):
            pltpu.make_async_copy(rows_v.at[b],
                                  acc.at[dst_v.at[0]], ss[b]).wait()

        for b in range(2 * GRP):
            fire_gather(b, b)

        def outer(g, carry):
            for bank in range(2):
                base = 2 * GRP * g + GRP * bank
                for i in range(GRP):
                    b = GRP * bank + i
                    wait_gather(b)
                    fire_scatter(base + i, b)
                for i in range(GRP):
                    b = GRP * bank + i
                    wait_scatter(b)
                    nxt = base + i + 2 * GRP

                    @pl.when(nxt < CPT)
                    def _():
                        fire_gather(nxt, b)
            return carry

        lax.fori_loop(0, CPT // (2 * GRP), outer, 0)
        plsc.subcore_barrier()
        pltpu.sync_copy(acc.at[pl.ds(s * RPT, RPT)],
                        out_hbm.at[c, pl.ds(s * RPT, RPT)])

    return k(table, src3, dst3, zeros)


def _relu(t):
    return jnp.maximum(t, 0.0)


def _dot(a, b):
    return jnp.dot(a, b, preferred_element_type=jnp.float32)


def _tc_first(x, w1, b1, w2, b2, w3, b3, cw):
    def body(x_r, w1_r, b1_r, w2_r, b2_r, w3_r, b3_r, cw_r,
             fca_r, fbc_r, hw_r):
        xv = x_r[...]
        fca_r[...] = _relu(_dot(xv, w1_r[...]) + b1_r[...])
        fbc_r[...] = (_relu(_dot(xv, w2_r[...]) + b2_r[...]) *
                      _relu(_dot(xv, w3_r[...]) + b3_r[...]))
        hw_r[...] = _dot(xv, cw_r[...])

    return pl.pallas_call(
        body,
        out_shape=[
            jax.ShapeDtypeStruct((N, 16), jnp.float32),
            jax.ShapeDtypeStruct((N, 16), jnp.float32),
            jax.ShapeDtypeStruct((N, F), jnp.float32),
        ],
    )(x, w1, b1, w2, b2, w3, b3, cw)


def _finish_layer(p0_r, p1_r, cb_r, fca_r, fbc_r, g_r, b_r):
    conv = _relu(p0_r[...] + p1_r[...] + cb_r[...])
    h = jnp.concatenate([fca_r[...], conv, fbc_r[...]], axis=1)
    m = jnp.mean(h, axis=0, keepdims=True)
    v = jnp.mean((h - m) * (h - m), axis=0, keepdims=True)
    return (h - m) * lax.rsqrt(v + 1e-5) * g_r[...] + b_r[...]


def _tc_mid(p0, p1, cb, fca, fbc, g, b, w1, b1, w2, b2, w3, b3, cw):
    def body(p0_r, p1_r, cb_r, fca_r, fbc_r, g_r, b_r,
             w1_r, b1_r, w2_r, b2_r, w3_r, b3_r, cw_r,
             fca_o, fbc_o, hw_o):
        h = _finish_layer(p0_r, p1_r, cb_r, fca_r, fbc_r, g_r, b_r)
        fca_o[...] = _relu(_dot(h, w1_r[...]) + b1_r[...])
        fbc_o[...] = (_relu(_dot(h, w2_r[...]) + b2_r[...]) *
                      _relu(_dot(h, w3_r[...]) + b3_r[...]))
        hw_o[...] = _dot(h, cw_r[...])

    return pl.pallas_call(
        body,
        out_shape=[
            jax.ShapeDtypeStruct((N, 16), jnp.float32),
            jax.ShapeDtypeStruct((N, 16), jnp.float32),
            jax.ShapeDtypeStruct((N, F), jnp.float32),
        ],
    )(p0, p1, cb, fca, fbc, g, b, w1, b1, w2, b2, w3, b3, cw)


def _tc_final(p0, p1, cb, fca, fbc, g, b, batch2d, fc1_w, fc1_b, fc2_w, fc2_b):
    def body(p0_r, p1_r, cb_r, fca_r, fbc_r, g_r, b_r, bt_r,
             w1_r, b1_r, w2_r, b2_r, out_r):
        h = _finish_layer(p0_r, p1_r, cb_r, fca_r, fbc_r, g_r, b_r)
        seg = lax.broadcasted_iota(jnp.int32, (N, NG), 1)
        oh = (bt_r[...] == seg).astype(jnp.float32)
        sums = lax.dot_general(oh, h, (((0,), (0,)), ((), ())),
                               preferred_element_type=jnp.float32)
        cnt = jnp.sum(oh, axis=0)
        pooled = sums / jnp.maximum(cnt, 1.0)[:, None]
        hid = _relu(_dot(pooled, w1_r[...]) + b1_r[...])
        out_r[...] = _dot(hid, w2_r[...]) + b2_r[...]

    return pl.pallas_call(
        body,
        out_shape=jax.ShapeDtypeStruct((NG, 1), jnp.float32),
    )(p0, p1, cb, fca, fbc, g, b, batch2d, fc1_w, fc1_b, fc2_w, fc2_b)


def kernel(x, edge_index, batch,
           fc11_w, fc11_b, fc12_w, fc12_b, fc13_w, fc13_b, conv11_w, conv11_b,
           bn1_g, bn1_b,
           fc21_w, fc21_b, fc22_w, fc22_b, fc23_w, fc23_b, conv21_w, conv21_b,
           bn2_g, bn2_b,
           fc31_w, fc31_b, fc32_w, fc32_b, fc33_w, fc33_b, conv31_w, conv31_b,
           bn3_g, bn3_b,
           fc1_w, fc1_b, fc2_w, fc2_b):
    r2 = lambda t: t[None, :]

    pad = E_PAD - E
    src = jnp.concatenate([edge_index[0], jnp.zeros((pad,), jnp.int32)])
    dst = jnp.concatenate([edge_index[1], jnp.full((pad,), DUMMY, jnp.int32)])
    src3 = src.reshape(NW, CPT, CHUNK)
    dst3 = dst.reshape(NW, CPT, CHUNK)
    zeros = jnp.zeros((N_ACC, F), jnp.float32)
    batch2d = batch.reshape(N, 1)

    fca, fbc, hw = _tc_first(x, fc11_w, r2(fc11_b), fc12_w, r2(fc12_b),
                             fc13_w, r2(fc13_b), conv11_w)
    agg = _sc_scatter_add(hw, src3, dst3, zeros)
    p0, p1 = agg[0, :N], agg[1, :N]

    fca, fbc, hw = _tc_mid(p0, p1, r2(conv11_b), fca, fbc, r2(bn1_g), r2(bn1_b),
                           fc21_w, r2(fc21_b), fc22_w, r2(fc22_b),
                           fc23_w, r2(fc23_b), conv21_w)
    agg = _sc_scatter_add(hw, src3, dst3, zeros)
    p0, p1 = agg[0, :N], agg[1, :N]

    fca, fbc, hw = _tc_mid(p0, p1, r2(conv21_b), fca, fbc, r2(bn2_g), r2(bn2_b),
                           fc31_w, r2(fc31_b), fc32_w, r2(fc32_b),
                           fc33_w, r2(fc33_b), conv31_w)
    agg = _sc_scatter_add(hw, src3, dst3, zeros)
    p0, p1 = agg[0, :N], agg[1, :N]

    return _tc_final(p0, p1, r2(conv31_b), fca, fbc, r2(bn3_g), r2(bn3_b),
                     batch2d, fc1_w, r2(fc1_b), fc2_w, r2(fc2_b))


# speedup vs baseline: 17.8615x; 2.1670x over previous
import functools

import jax
import jax.numpy as jnp
from jax import lax
from jax.experimental import pallas as pl
from jax.experimental.pallas import tpu as pltpu
from jax.experimental.pallas import tpu_sc as plsc

N = 10000
E = 320000
NG = 64
F = 32
NC, NS = 2, 16
NW = NC * NS
CHUNK = 128
GRP = 4
CPT = 80
E_PAD = NW * CHUNK * CPT
N_ACC = 10112
DUMMY = N
RPT = N_ACC // NS


def _sc_scatter_add(table, src3, dst3, zeros):
    mesh = plsc.VectorSubcoreMesh(core_axis_name="c", subcore_axis_name="s")

    @functools.partial(
        pl.kernel,
        out_type=jax.ShapeDtypeStruct((NC, N_ACC, F), jnp.float32),
        mesh=mesh,
        scratch_types=[
            pltpu.VMEM_SHARED((N_ACC, F), jnp.float32),
            pltpu.VMEM((CPT, CHUNK), jnp.int32),
            pltpu.VMEM((CPT, CHUNK), jnp.int32),
            pltpu.VMEM((2 * GRP, CHUNK, F), jnp.float32),
            [pltpu.SemaphoreType.DMA] * (2 * GRP),
            [pltpu.SemaphoreType.DMA] * (2 * GRP),
        ],
        compiler_params=pltpu.CompilerParams(use_tc_tiling_on_sc=False),
    )
    def k(table_hbm, src_hbm, dst_hbm, zero_hbm, out_hbm,
          acc, src_v, dst_v, rows_v, sg, ss):
        c = lax.axis_index("c")
        s = lax.axis_index("s")
        glob = c * NS + s
        pltpu.sync_copy(zero_hbm.at[pl.ds(s * RPT, RPT)],
                        acc.at[pl.ds(s * RPT, RPT)])
        pltpu.sync_copy(src_hbm.at[glob], src_v)
        pltpu.sync_copy(dst_hbm.at[glob], dst_v)
        plsc.subcore_barrier()

        def fire_gather(j, b):
            pltpu.async_copy(table_hbm.at[src_v.at[j]], rows_v.at[b], sg[b])

        def wait_gather(b):
            pltpu.make_async_copy(table_hbm.at[src_v.at[0]],
                                  rows_v.at[b], sg[b]).wait()

        def fire_scatter(j, b):
            pltpu.async_copy(rows_v.at[b], acc.at[dst_v.at[j]], ss[b],
                             add=True)

        def wait_scatter(b):
            pltpu.make_async_copy(rows_v.at[b],
                                  acc.at[dst_v.at[0]], ss[b]).wait()

        for b in range(2 * GRP):
            fire_gather(b, b)

        def outer(g, carry):
            for bank in range(2):
                base = 2 * GRP * g + GRP * bank
                for i in range(GRP):
                    b = GRP * bank + i
                    wait_gather(b)
                    fire_scatter(base + i, b)
                for i in range(GRP):
                    b = GRP * bank + i
                    wait_scatter(b)
                    nxt = base + i + 2 * GRP

                    @pl.when(nxt < CPT)
                    def _():
                        fire_gather(nxt, b)
            return carry

        lax.fori_loop(0, CPT // (2 * GRP), outer, 0)
        plsc.subcore_barrier()
        pltpu.sync_copy(acc.at[pl.ds(s * RPT, RPT)],
                        out_hbm.at[c, pl.ds(s * RPT, RPT)])

    return k(table, src3, dst3, zeros)


def _relu(t):
    return jnp.maximum(t, 0.0)


def _dot(a, b):
    return jnp.dot(a, b, preferred_element_type=jnp.float32)


def _tc_first(x, w1, b1, w2, b2, w3, b3, cw):
    def body(x_r, w1_r, b1_r, w2_r, b2_r, w3_r, b3_r, cw_r,
             fca_r, fbc_r, hw_r):
        xv = x_r[...]
        fca_r[...] = _relu(_dot(xv, w1_r[...]) + b1_r[...])
        fbc_r[...] = (_relu(_dot(xv, w2_r[...]) + b2_r[...]) *
                      _relu(_dot(xv, w3_r[...]) + b3_r[...]))
        hw_r[...] = _dot(xv, cw_r[...])

    return pl.pallas_call(
        body,
        out_shape=[
            jax.ShapeDtypeStruct((N, 16), jnp.float32),
            jax.ShapeDtypeStruct((N, 16), jnp.float32),
            jax.ShapeDtypeStruct((N, F), jnp.float32),
        ],
    )(x, w1, b1, w2, b2, w3, b3, cw)


def _finish_layer(p0_r, p1_r, cb_r, fca_r, fbc_r, g_r, b_r):
    conv = _relu(p0_r[...] + p1_r[...] + cb_r[...])
    h = jnp.concatenate([fca_r[...], conv, fbc_r[...]], axis=1)
    m = jnp.mean(h, axis=0, keepdims=True)
    v = jnp.mean((h - m) * (h - m), axis=0, keepdims=True)
    return (h - m) * lax.rsqrt(v + 1e-5) * g_r[...] + b_r[...]


def _tc_mid(p0, p1, cb, fca, fbc, g, b, w1, b1, w2, b2, w3, b3, cw):
    def body(p0_r, p1_r, cb_r, fca_r, fbc_r, g_r, b_r,
             w1_r, b1_r, w2_r, b2_r, w3_r, b3_r, cw_r,
             fca_o, fbc_o, hw_o):
        h = _finish_layer(p0_r, p1_r, cb_r, fca_r, fbc_r, g_r, b_r)
        fca_o[...] = _relu(_dot(h, w1_r[...]) + b1_r[...])
        fbc_o[...] = (_relu(_dot(h, w2_r[...]) + b2_r[...]) *
                      _relu(_dot(h, w3_r[...]) + b3_r[...]))
        hw_o[...] = _dot(h, cw_r[...])

    return pl.pallas_call(
        body,
        out_shape=[
            jax.ShapeDtypeStruct((N, 16), jnp.float32),
            jax.ShapeDtypeStruct((N, 16), jnp.float32),
            jax.ShapeDtypeStruct((N, F), jnp.float32),
        ],
    )(p0, p1, cb, fca, fbc, g, b, w1, b1, w2, b2, w3, b3, cw)


def _tc_final(p0, p1, cb, fca, fbc, g, b, batch2d, fc1_w, fc1_b, fc2_w, fc2_b):
    def body(p0_r, p1_r, cb_r, fca_r, fbc_r, g_r, b_r, bt_r,
             w1_r, b1_r, w2_r, b2_r, out_r):
        h = _finish_layer(p0_r, p1_r, cb_r, fca_r, fbc_r, g_r, b_r)
        seg = lax.broadcasted_iota(jnp.int32, (N, NG), 1)
        oh = (bt_r[...] == seg).astype(jnp.float32)
        sums = lax.dot_general(oh, h, (((0,), (0,)), ((), ())),
                               preferred_element_type=jnp.float32)
        cnt = jnp.sum(oh, axis=0)
        pooled = sums / jnp.maximum(cnt, 1.0)[:, None]
        hid = _relu(_dot(pooled, w1_r[...]) + b1_r[...])
        out_r[...] = _dot(hid, w2_r[...]) + b2_r[...]

    return pl.pallas_call(
        body,
        out_shape=jax.ShapeDtypeStruct((NG, 1), jnp.float32),
    )(p0, p1, cb, fca, fbc, g, b, batch2d, fc1_w, fc1_b, fc2_w, fc2_b)


def kernel(x, edge_index, batch,
           fc11_w, fc11_b, fc12_w, fc12_b, fc13_w, fc13_b, conv11_w, conv11_b,
           bn1_g, bn1_b,
           fc21_w, fc21_b, fc22_w, fc22_b, fc23_w, fc23_b, conv21_w, conv21_b,
           bn2_g, bn2_b,
           fc31_w, fc31_b, fc32_w, fc32_b, fc33_w, fc33_b, conv31_w, conv31_b,
           bn3_g, bn3_b,
           fc1_w, fc1_b, fc2_w, fc2_b):
    r2 = lambda t: t[None, :]

    pad = E_PAD - E
    ar = jnp.arange(pad, dtype=jnp.int32)
    src = jnp.concatenate([edge_index[0], ar % N])
    dst = jnp.concatenate([edge_index[1], DUMMY + ar % (N_ACC - N)])
    src3 = src.reshape(NW, CPT, CHUNK)
    dst3 = dst.reshape(NW, CPT, CHUNK)
    zeros = jnp.zeros((N_ACC, F), jnp.float32)
    batch2d = batch.reshape(N, 1)

    fca, fbc, hw = _tc_first(x, fc11_w, r2(fc11_b), fc12_w, r2(fc12_b),
                             fc13_w, r2(fc13_b), conv11_w)
    agg = _sc_scatter_add(hw, src3, dst3, zeros)
    p0, p1 = agg[0, :N], agg[1, :N]

    fca, fbc, hw = _tc_mid(p0, p1, r2(conv11_b), fca, fbc, r2(bn1_g), r2(bn1_b),
                           fc21_w, r2(fc21_b), fc22_w, r2(fc22_b),
                           fc23_w, r2(fc23_b), conv21_w)
    agg = _sc_scatter_add(hw, src3, dst3, zeros)
    p0, p1 = agg[0, :N], agg[1, :N]

    fca, fbc, hw = _tc_mid(p0, p1, r2(conv21_b), fca, fbc, r2(bn2_g), r2(bn2_b),
                           fc31_w, r2(fc31_b), fc32_w, r2(fc32_b),
                           fc33_w, r2(fc33_b), conv31_w)
    agg = _sc_scatter_add(hw, src3, dst3, zeros)
    p0, p1 = agg[0, :N], agg[1, :N]

    return _tc_final(p0, p1, r2(conv31_b), fca, fbc, r2(bn3_g), r2(bn3_b),
                     batch2d, fc1_w, r2(fc1_b), fc2_w, r2(fc2_b))


# speedup vs baseline: 19.4823x; 1.0907x over previous
import functools

import jax
import jax.numpy as jnp
from jax import lax
from jax.experimental import pallas as pl
from jax.experimental.pallas import tpu as pltpu
from jax.experimental.pallas import tpu_sc as plsc

N = 10000
E = 320000
NG = 64
F = 32
NC, NS = 2, 16
NW = NC * NS
CHUNK = 128
GRP = 4
CPT = 80
E_PAD = NW * CHUNK * CPT
N_ACC = 10112
DUMMY = N
RPT = N_ACC // NS


def _sc_scatter_add(table, src3, dst3, zeros):
    mesh = plsc.VectorSubcoreMesh(core_axis_name="c", subcore_axis_name="s")

    @functools.partial(
        pl.kernel,
        out_type=jax.ShapeDtypeStruct((NC, N_ACC, F), jnp.float32),
        mesh=mesh,
        scratch_types=[
            pltpu.VMEM_SHARED((N_ACC, F), jnp.float32),
            pltpu.VMEM((CPT, CHUNK), jnp.int32),
            pltpu.VMEM((CPT, CHUNK), jnp.int32),
            pltpu.VMEM((2 * GRP, CHUNK, F), jnp.float32),
            [pltpu.SemaphoreType.DMA] * (2 * GRP),
            [pltpu.SemaphoreType.DMA] * (2 * GRP),
        ],
        compiler_params=pltpu.CompilerParams(use_tc_tiling_on_sc=False),
    )
    def k(table_hbm, src_hbm, dst_hbm, zero_hbm, out_hbm,
          acc, src_v, dst_v, rows_v, sg, ss):
        c = lax.axis_index("c")
        s = lax.axis_index("s")
        glob = c * NS + s
        pltpu.sync_copy(zero_hbm.at[pl.ds(s * RPT, RPT)],
                        acc.at[pl.ds(s * RPT, RPT)])
        pltpu.sync_copy(src_hbm.at[glob], src_v)
        pltpu.sync_copy(dst_hbm.at[glob], dst_v)
        plsc.subcore_barrier()

        def fire_gather(j, b):
            pltpu.async_copy(table_hbm.at[src_v.at[j]], rows_v.at[b], sg[b])

        def wait_gather(b):
            pltpu.make_async_copy(table_hbm.at[src_v.at[0]],
                                  rows_v.at[b], sg[b]).wait()

        def fire_scatter(j, b):
            pltpu.async_copy(rows_v.at[b], acc.at[dst_v.at[j]], ss[b],
                             add=True)

        def wait_scatter(b):
            pltpu.make_async_copy(rows_v.at[b],
                                  acc.at[dst_v.at[0]], ss[b]).wait()

        for b in range(2 * GRP):
            fire_gather(b, b)

        def outer(g, carry):
            for bank in range(2):
                base = 2 * GRP * g + GRP * bank
                for i in range(GRP):
                    b = GRP * bank + i
                    wait_gather(b)
                    fire_scatter(base + i, b)
                for i in range(GRP):
                    b = GRP * bank + i
                    wait_scatter(b)
                    nxt = base + i + 2 * GRP

                    @pl.when(nxt < CPT)
                    def _():
                        fire_gather(nxt, b)
            return carry

        lax.fori_loop(0, CPT // (2 * GRP), outer, 0)
        plsc.subcore_barrier()
        pltpu.sync_copy(acc.at[pl.ds(s * RPT, RPT)],
                        out_hbm.at[c, pl.ds(s * RPT, RPT)])

    return k(table, src3, dst3, zeros)


def _relu(t):
    return jnp.maximum(t, 0.0)


def _dot(a, b):
    return jnp.dot(a, b, preferred_element_type=jnp.float32)


def _split_dense(hc, b1_r, b2_r, b3_r):
    fca = _relu(hc[:, 0:16] + b1_r[...])
    fbc = (_relu(hc[:, 16:32] + b2_r[...]) *
           _relu(hc[:, 32:48] + b3_r[...]))
    return fca, fbc, hc[:, 48:80]


def _tc_first(x, wcat, b1, b2, b3):
    def body(x_r, w_r, b1_r, b2_r, b3_r, fca_r, fbc_r, hw_r):
        hc = _dot(x_r[...], w_r[...])
        fca_r[...], fbc_r[...], hw_r[...] = _split_dense(hc, b1_r, b2_r, b3_r)

    return pl.pallas_call(
        body,
        out_shape=[
            jax.ShapeDtypeStruct((N, 16), jnp.float32),
            jax.ShapeDtypeStruct((N, 16), jnp.float32),
            jax.ShapeDtypeStruct((N, F), jnp.float32),
        ],
    )(x, wcat, b1, b2, b3)


def _finish_layer(agg_r, cb_r, fca_r, fbc_r, g_r, b_r):
    conv = _relu(agg_r[0, :N, :] + agg_r[1, :N, :] + cb_r[...])
    h = jnp.concatenate([fca_r[...], conv, fbc_r[...]], axis=1)
    m = jnp.mean(h, axis=0, keepdims=True)
    v = jnp.mean((h - m) * (h - m), axis=0, keepdims=True)
    return (h - m) * lax.rsqrt(v + 1e-5) * g_r[...] + b_r[...]


def _tc_mid(agg, cb, fca, fbc, g, b, wcat, b1, b2, b3):
    def body(agg_r, cb_r, fca_r, fbc_r, g_r, b_r,
             w_r, b1_r, b2_r, b3_r,
             fca_o, fbc_o, hw_o):
        h = _finish_layer(agg_r, cb_r, fca_r, fbc_r, g_r, b_r)
        hc = _dot(h, w_r[...])
        fca_o[...], fbc_o[...], hw_o[...] = _split_dense(hc, b1_r, b2_r, b3_r)

    return pl.pallas_call(
        body,
        out_shape=[
            jax.ShapeDtypeStruct((N, 16), jnp.float32),
            jax.ShapeDtypeStruct((N, 16), jnp.float32),
            jax.ShapeDtypeStruct((N, F), jnp.float32),
        ],
    )(agg, cb, fca, fbc, g, b, wcat, b1, b2, b3)


def _tc_final(agg, cb, fca, fbc, g, b, batch2d, fc1_w, fc1_b, fc2_w, fc2_b):
    def body(agg_r, cb_r, fca_r, fbc_r, g_r, b_r, bt_r,
             w1_r, b1_r, w2_r, b2_r, out_r):
        h = _finish_layer(agg_r, cb_r, fca_r, fbc_r, g_r, b_r)
        seg = lax.broadcasted_iota(jnp.int32, (N, NG), 1)
        oh = (bt_r[...] == seg).astype(jnp.float32)
        sums = lax.dot_general(oh, h, (((0,), (0,)), ((), ())),
                               preferred_element_type=jnp.float32)
        cnt = jnp.sum(oh, axis=0)
        pooled = sums / jnp.maximum(cnt, 1.0)[:, None]
        hid = _relu(_dot(pooled, w1_r[...]) + b1_r[...])
        out_r[...] = _dot(hid, w2_r[...]) + b2_r[...]

    return pl.pallas_call(
        body,
        out_shape=jax.ShapeDtypeStruct((NG, 1), jnp.float32),
    )(agg, cb, fca, fbc, g, b, batch2d, fc1_w, fc1_b, fc2_w, fc2_b)


def kernel(x, edge_index, batch,
           fc11_w, fc11_b, fc12_w, fc12_b, fc13_w, fc13_b, conv11_w, conv11_b,
           bn1_g, bn1_b,
           fc21_w, fc21_b, fc22_w, fc22_b, fc23_w, fc23_b, conv21_w, conv21_b,
           bn2_g, bn2_b,
           fc31_w, fc31_b, fc32_w, fc32_b, fc33_w, fc33_b, conv31_w, conv31_b,
           bn3_g, bn3_b,
           fc1_w, fc1_b, fc2_w, fc2_b):
    r2 = lambda t: t[None, :]

    pad = E_PAD - E
    ar = jnp.arange(pad, dtype=jnp.int32)
    src = jnp.concatenate([edge_index[0], ar % N])
    dst = jnp.concatenate([edge_index[1], DUMMY + ar % (N_ACC - N)])
    src3 = src.reshape(NW, CPT, CHUNK)
    dst3 = dst.reshape(NW, CPT, CHUNK)
    zeros = jnp.zeros((N_ACC, F), jnp.float32)
    batch2d = batch.reshape(N, 1)
    wcat1 = jnp.concatenate([fc11_w, fc12_w, fc13_w, conv11_w], axis=1)
    wcat2 = jnp.concatenate([fc21_w, fc22_w, fc23_w, conv21_w], axis=1)
    wcat3 = jnp.concatenate([fc31_w, fc32_w, fc33_w, conv31_w], axis=1)

    fca, fbc, hw = _tc_first(x, wcat1, r2(fc11_b), r2(fc12_b), r2(fc13_b))
    agg = _sc_scatter_add(hw, src3, dst3, zeros)

    fca, fbc, hw = _tc_mid(agg, r2(conv11_b), fca, fbc, r2(bn1_g), r2(bn1_b),
                           wcat2, r2(fc21_b), r2(fc22_b), r2(fc23_b))
    agg = _sc_scatter_add(hw, src3, dst3, zeros)

    fca, fbc, hw = _tc_mid(agg, r2(conv21_b), fca, fbc, r2(bn2_g), r2(bn2_b),
                           wcat3, r2(fc31_b), r2(fc32_b), r2(fc33_b))
    agg = _sc_scatter_add(hw, src3, dst3, zeros)

    return _tc_final(agg, r2(conv31_b), fca, fbc, r2(bn3_g), r2(bn3_b),
                     batch2d, fc1_w, r2(fc1_b), fc2_w, r2(fc2_b))
